# baseline (speedup 1.0000x reference)
import jax
import jax.numpy as jnp
import numpy as np
from jax.experimental import pallas as pl
from jax.scipy.special import gammaln

N = 10000
E = 160000
F = 128
K = 32
G = 64
CUTOFF = 5.0


def _smooth_cutoff(r, cutoff):
    x = r / cutoff
    x2 = jnp.clip(x, 0.0, 1.0 - 1e-6) ** 2
    f = jnp.exp(1.0 - 1.0 / (1.0 - x2))
    return jnp.where(x < 1.0, f, 0.0)


def _reciprocal_bernstein(r, num):
    u = 1.0 / (1.0 + r)
    k = jnp.arange(num, dtype=jnp.float32)
    n = float(num - 1)
    log_binom = gammaln(n + 1.0) - gammaln(k + 1.0) - gammaln(n - k + 1.0)
    log_u = jnp.log(jnp.clip(u, 1e-10, 1.0))[:, None]
    log_1mu = jnp.log(jnp.clip(1.0 - u, 1e-10, 1.0))[:, None]
    return jnp.exp(log_binom[None, :] + k[None, :] * log_u + (n - k)[None, :] * log_1mu)


def _deloc_embed(e_Z, psi, batch_segments, graph_mask, p):
    q = e_Z @ p['Wq']
    k_graph = jnp.where((psi >= 0.0)[:, None], p['k_pos'][None, :], p['k_neg'][None, :])
    k_at = k_graph[batch_segments]
    wgt = jax.nn.softplus(jnp.sum(q * k_at, axis=-1) / np.sqrt(F))
    denom = jax.ops.segment_sum(wgt, batch_segments, num_segments=G)
    wn = wgt / (denom[batch_segments] + 1e-8)
    e = (psi[batch_segments] * wn)[:, None] * (e_Z @ p['Wv'])
    return jnp.where(graph_mask[batch_segments][:, None], e, 0.0)


def _tensor_dense(y, p, max_degree):
    u0 = y[:, :1] @ p['Wu0']; u1 = y[:, 1:] @ p['Wu1']
    v0 = y[:, :1] @ p['Wv0']; v1 = y[:, 1:] @ p['Wv1']
    t0 = u0 * v0 + jnp.sum(u1 * v1, axis=1, keepdims=True)
    o0 = t0 @ p['Wo0'] + p['bo0']
    if max_degree == 0:
        return o0
    t1 = u0 * v1 + u1 * v0 + jnp.cross(u1, v1, axisa=1, axisb=1, axisc=1)
    o1 = t1 @ p['Wo1']
    return jnp.concatenate([o0, o1], axis=1)


def _copy_body(x_ref, o_ref):
    o_ref[...] = x_ref[...]


def kernel(positions, atomic_numbers, dst_idx, src_idx, num_unpaired_electrons,
           total_charge, batch_segments, graph_mask, params):
    pos_dst = positions[dst_idx]
    pos_src = positions[src_idx]
    disp = pos_src - pos_dst
    r = jnp.sqrt(jnp.sum(disp * disp, axis=-1) + 1e-12)
    radial = _reciprocal_bernstein(r, K) * _smooth_cutoff(r, CUTOFF)[:, None]
    unit = disp / r[:, None]
    sph = jnp.concatenate([jnp.ones((disp.shape[0], 1), jnp.float32), unit], axis=1)
    basis = sph[:, :, None] * radial[:, None, :]
    e_Z = params['embed'][atomic_numbers]
    e_Q = _deloc_embed(e_Z, total_charge, batch_segments, graph_mask, params['Q'])
    e_S = _deloc_embed(e_Z, num_unpaired_electrons, batch_segments, graph_mask, params['S'])
    x = e_Z + e_Q + e_S
    x_src = x[src_idx]
    y_lms = []
    for lm in range(4):
        bp = basis[:, lm, :] @ params['W_mp']
        y_lms.append(jax.ops.segment_sum(bp * x_src, dst_idx, num_segments=N))
    y = jnp.stack(y_lms, axis=1)
    z = x @ params['Wz'] + params['bz']
    y = y.at[:, 0, :].add(z)
    y0 = y[:, :1] @ params['Wd0'] + params['bd0']
    y1 = y[:, 1:] @ params['Wd1']
    y = jnp.concatenate([y0, y1], axis=1)
    y = y + _tensor_dense(y, params['itp0'], 1)
    out0 = y[:, :1] + _tensor_dense(y, params['itp1'], 0)
    flat = out0.reshape(N, F)
    flat = pl.pallas_call(
        _copy_body,
        out_shape=jax.ShapeDtypeStruct((N, F), jnp.float32),
    )(flat)
    return flat[:, None, None, :]


# speedup vs baseline: 2.6177x; 2.6177x over previous
import functools
import math

import jax
import jax.numpy as jnp
import numpy as np
from jax import lax
from jax.experimental import pallas as pl
from jax.experimental.pallas import tpu as pltpu
from jax.experimental.pallas import tpu_sc as plsc

N = 10000
E = 160000
F = 128
K = 32
G = 64
CUTOFF = 5.0

NC = 2
NS = 16
NW = NC * NS

_HIGH = jax.lax.Precision.HIGHEST


def _dot(a, b):
    return jnp.dot(a, b, precision=_HIGH, preferred_element_type=jnp.float32)


_CN = 1000
_GN = N // _CN


def _b1_body(an_ref, seg_ref, embed_ref, wq_q, kp_q, kn_q, psi_q,
             wq_s, kp_s, kn_s, psi_s,
             wgtq_ref, wgts_ref, denq_ref, dens_ref):
    i = pl.program_id(0)
    an = an_ref[...]
    onez = (an == lax.broadcasted_iota(jnp.int32, (_CN, 128), 1)).astype(jnp.float32)
    e_z = _dot(onez, embed_ref[...])
    seg = seg_ref[...]
    segoh = (seg == lax.broadcasted_iota(jnp.int32, (_CN, G), 1)).astype(jnp.float32)

    @pl.when(i == 0)
    def _():
        denq_ref[...] = jnp.zeros_like(denq_ref)
        dens_ref[...] = jnp.zeros_like(dens_ref)

    def logits(wq, kp, kn, psi):
        q = _dot(e_z, wq[...])
        psi_at = _dot(segoh, psi[...])
        mpos = (psi_at >= 0.0).astype(jnp.float32)
        k_at = mpos * kp[...] + (1.0 - mpos) * kn[...]
        s = jnp.sum(q * k_at, axis=1, keepdims=True) * (1.0 / np.sqrt(F))
        return jax.nn.softplus(s)

    wgt_q = logits(wq_q, kp_q, kn_q, psi_q)
    wgt_s = logits(wq_s, kp_s, kn_s, psi_s)
    wgtq_ref[...] = wgt_q
    wgts_ref[...] = wgt_s
    denq_ref[...] += lax.dot_general(segoh, wgt_q, (((0,), (0,)), ((), ())),
                                     precision=_HIGH, preferred_element_type=jnp.float32)
    dens_ref[...] += lax.dot_general(segoh, wgt_s, (((0,), (0,)), ((), ())),
                                     precision=_HIGH, preferred_element_type=jnp.float32)


def _node_logits(an2, seg2, embed_p, pQ, pS, psiQ, psiS):
    full = lambda s: pl.BlockSpec(s, lambda i: (0, 0))
    return pl.pallas_call(
        _b1_body,
        grid=(_GN,),
        in_specs=[
            pl.BlockSpec((_CN, 1), lambda i: (i, 0)),
            pl.BlockSpec((_CN, 1), lambda i: (i, 0)),
            full((128, 128)),
            full((128, 128)), full((1, 128)), full((1, 128)), full((G, 1)),
            full((128, 128)), full((1, 128)), full((1, 128)), full((G, 1)),
        ],
        out_specs=[
            pl.BlockSpec((_CN, 1), lambda i: (i, 0)),
            pl.BlockSpec((_CN, 1), lambda i: (i, 0)),
            full((G, 1)), full((G, 1)),
        ],
        out_shape=[
            jax.ShapeDtypeStruct((N, 1), jnp.float32),
            jax.ShapeDtypeStruct((N, 1), jnp.float32),
            jax.ShapeDtypeStruct((G, 1), jnp.float32),
            jax.ShapeDtypeStruct((G, 1), jnp.float32),
        ],
    )(an2, seg2, embed_p, pQ['Wq'], pQ['k_pos'][None, :], pQ['k_neg'][None, :], psiQ,
      pS['Wq'], pS['k_pos'][None, :], pS['k_neg'][None, :], psiS)


def _b2_body(an_ref, seg_ref, embed_ref, wv_q, wv_s, wgtq_ref, wgts_ref,
             denq_ref, dens_ref, psiq_ref, psis_ref, maskf_ref, x_ref):
    an = an_ref[...]
    onez = (an == lax.broadcasted_iota(jnp.int32, (_CN, 128), 1)).astype(jnp.float32)
    e_z = _dot(onez, embed_ref[...])
    seg = seg_ref[...]
    segoh = (seg == lax.broadcasted_iota(jnp.int32, (_CN, G), 1)).astype(jnp.float32)
    mask_at = _dot(segoh, maskf_ref[...])

    def deloc(wv, wgt, den, psi):
        den_at = _dot(segoh, den[...])
        psi_at = _dot(segoh, psi[...])
        wn = wgt[...] / (den_at + 1e-8)
        coef = psi_at * wn * mask_at
        return coef * _dot(e_z, wv[...])

    x_ref[...] = (e_z + deloc(wv_q, wgtq_ref, denq_ref, psiq_ref)
                  + deloc(wv_s, wgts_ref, dens_ref, psis_ref))


def _node_features(an2, seg2, embed_p, pQ, pS, wgtQ, wgtS, denQ, denS,
                   psiQ, psiS, maskf):
    full = lambda s: pl.BlockSpec(s, lambda i: (0, 0))
    return pl.pallas_call(
        _b2_body,
        grid=(_GN,),
        in_specs=[
            pl.BlockSpec((_CN, 1), lambda i: (i, 0)),
            pl.BlockSpec((_CN, 1), lambda i: (i, 0)),
            full((128, 128)), full((128, 128)), full((128, 128)),
            pl.BlockSpec((_CN, 1), lambda i: (i, 0)),
            pl.BlockSpec((_CN, 1), lambda i: (i, 0)),
            full((G, 1)), full((G, 1)), full((G, 1)), full((G, 1)), full((G, 1)),
        ],
        out_specs=pl.BlockSpec((_CN, 128), lambda i: (i, 0)),
        out_shape=jax.ShapeDtypeStruct((N, 128), jnp.float32),
    )(an2, seg2, embed_p, pQ['Wv'], pS['Wv'], wgtQ, wgtS, denQ, denS,
      psiQ, psiS, maskf)


_EPW = E // NW
_GC = 128
_GFULL = _EPW // _GC
_GTAIL = _EPW - _GFULL * _GC

_sc_mesh = plsc.VectorSubcoreMesh(core_axis_name="c", subcore_axis_name="s")


@functools.partial(
    pl.kernel,
    out_type=(
        jax.ShapeDtypeStruct((E * 16,), jnp.float32),
        jax.ShapeDtypeStruct((E, 128), jnp.float32),
    ),
    mesh=_sc_mesh,
    scratch_types=[
        pltpu.VMEM((_GC,), jnp.int32),
        pltpu.VMEM((_GC,), jnp.int32),
        pltpu.VMEM((_GC, 128), jnp.float32),
        pltpu.VMEM((_GC, 128), jnp.float32),
        pltpu.VMEM((_GC, 128), jnp.float32),
        pltpu.VMEM((_GC * 16,), jnp.float32),
        pltpu.VMEM((_GTAIL,), jnp.int32),
        pltpu.VMEM((_GTAIL,), jnp.int32),
        pltpu.VMEM((_GTAIL, 128), jnp.float32),
        pltpu.VMEM((_GTAIL, 128), jnp.float32),
        pltpu.VMEM((_GTAIL, 128), jnp.float32),
        pltpu.VMEM((_GTAIL * 16,), jnp.float32),
        pltpu.SemaphoreType.DMA,
    ],
)
def _sc_gather(pos_hbm, x_hbm, dst_hbm, src_hbm,
               disp_out, xs_out,
               idxd_v, idxs_v, pd_v, ps_v, xs_v, dv_v,
               idxd_t, idxs_t, pd_t, ps_t, xs_t, dv_t, sem):
    wid = lax.axis_index("s") * NC + lax.axis_index("c")
    base = wid * _EPW

    def chunk(i, _):
        e0 = base + i * _GC
        pltpu.sync_copy(dst_hbm.at[pl.ds(e0, _GC)], idxd_v)
        pltpu.sync_copy(src_hbm.at[pl.ds(e0, _GC)], idxs_v)
        c1 = pltpu.async_copy(pos_hbm.at[idxd_v], pd_v, sem)
        c2 = pltpu.async_copy(pos_hbm.at[idxs_v], ps_v, sem)
        c3 = pltpu.async_copy(x_hbm.at[idxs_v], xs_v, sem)
        c1.wait(); c2.wait(); c3.wait()

        def drow(j, _):
            dv_v[pl.ds(j * 16, 16)] = ps_v[j, pl.ds(0, 16)] - pd_v[j, pl.ds(0, 16)]
            return 0

        lax.fori_loop(0, _GC, drow, 0)
        pltpu.sync_copy(dv_v, disp_out.at[pl.ds(e0 * 16, _GC * 16)])
        pltpu.sync_copy(xs_v, xs_out.at[pl.ds(e0, _GC)])
        return 0

    lax.fori_loop(0, _GFULL, chunk, 0)
    e0 = base + _GFULL * _GC
    pltpu.sync_copy(dst_hbm.at[pl.ds(e0, _GTAIL)], idxd_t)
    pltpu.sync_copy(src_hbm.at[pl.ds(e0, _GTAIL)], idxs_t)
    c1 = pltpu.async_copy(pos_hbm.at[idxd_t], pd_t, sem)
    c2 = pltpu.async_copy(pos_hbm.at[idxs_t], ps_t, sem)
    c3 = pltpu.async_copy(x_hbm.at[idxs_t], xs_t, sem)
    c1.wait(); c2.wait(); c3.wait()

    def drow_t(j, _):
        dv_t[pl.ds(j * 16, 16)] = ps_t[j, pl.ds(0, 16)] - pd_t[j, pl.ds(0, 16)]
        return 0

    lax.fori_loop(0, _GTAIL, drow_t, 0)
    pltpu.sync_copy(dv_t, disp_out.at[pl.ds(e0 * 16, _GTAIL * 16)])
    pltpu.sync_copy(xs_t, xs_out.at[pl.ds(e0, _GTAIL)])


_CE = 640
_GE = E // _CE

_LOG_BINOM = np.array(
    [math.lgamma(K) - math.lgamma(k + 1.0) - math.lgamma(K - 1.0 - k + 1.0)
     for k in range(K)], dtype=np.float32)[None, :]
_KARR = np.arange(K, dtype=np.float32)[None, :]


def _d_body(dsp_ref, xs_ref, wmp_ref, lb_ref, ka_ref,
            m0_ref, m1_ref, m2_ref, m3_ref):
    disp = dsp_ref[...]
    r2 = jnp.sum(disp * disp, axis=1, keepdims=True) + 1e-12
    r = jnp.sqrt(r2)
    u = 1.0 / (1.0 + r)
    log_u = jnp.log(jnp.clip(u, 1e-10, 1.0))
    log_1mu = jnp.log(jnp.clip(1.0 - u, 1e-10, 1.0))
    kb = ka_ref[...]
    radial = jnp.exp(lb_ref[...] + kb * log_u + (K - 1.0 - kb) * log_1mu)
    xx = r * (1.0 / CUTOFF)
    x2 = jnp.clip(xx, 0.0, 1.0 - 1e-6) ** 2
    cut = jnp.where(xx < 1.0, jnp.exp(1.0 - 1.0 / (1.0 - x2)), 0.0)
    radial = radial * cut
    rp = _dot(radial, wmp_ref[...])
    g = rp * xs_ref[...]
    inv_r = 1.0 / r
    m0_ref[...] = g
    m1_ref[...] = (disp[:, 0:1] * inv_r) * g
    m2_ref[...] = (disp[:, 1:2] * inv_r) * g
    m3_ref[...] = (disp[:, 2:3] * inv_r) * g


def _edge_messages(dsp, xs, wmp):
    eb = lambda w: pl.BlockSpec((_CE, w), lambda i: (i, 0))
    return pl.pallas_call(
        _d_body,
        grid=(_GE,),
        in_specs=[eb(16), eb(128), pl.BlockSpec((32, 128), lambda i: (0, 0)),
                  pl.BlockSpec((1, 32), lambda i: (0, 0)),
                  pl.BlockSpec((1, 32), lambda i: (0, 0))],
        out_specs=[eb(128)] * 4,
        out_shape=[jax.ShapeDtypeStruct((E, 128), jnp.float32)] * 4,
    )(dsp, xs, wmp, jnp.asarray(_LOG_BINOM), jnp.asarray(_KARR))


_EPT = E // NS
_SFULL = _EPT // _GC
_STAIL = _EPT - _SFULL * _GC
NPAD = 10240
_RPT = NPAD // NS
_RQ = 128


@functools.partial(
    pl.kernel,
    out_type=tuple(jax.ShapeDtypeStruct((NPAD, 128), jnp.float32) for _ in range(4)),
    mesh=_sc_mesh,
    scratch_types=[
        pltpu.VMEM((_GC, 128), jnp.float32),
        pltpu.VMEM((_GC,), jnp.int32),
        pltpu.VMEM((_STAIL, 128), jnp.float32),
        pltpu.VMEM((_STAIL,), jnp.int32),
        pltpu.VMEM_SHARED((NPAD, 128), jnp.float32),
    ],
)
def _sc_scatter(m0_hbm, m1_hbm, m2_hbm, m3_hbm, dst_hbm,
                y0_hbm, y1_hbm, y2_hbm, y3_hbm,
                m_v, idx_v, mt_v, idxt_v, acc):
    cid = lax.axis_index("c")
    sid = lax.axis_index("s")
    m_refs = (m0_hbm, m1_hbm, m2_hbm, m3_hbm)
    y_refs = (y0_hbm, y1_hbm, y2_hbm, y3_hbm)

    def one_pass(m_hbm, y_hbm):
        zv = jnp.zeros((16,), jnp.float32)

        def zrow(i, _):
            for j in range(8):
                m_v[i, pl.ds(j * 16, 16)] = zv
            return 0

        lax.fori_loop(0, _GC, zrow, 0)
        for q in range(_RPT // _RQ):
            pltpu.sync_copy(m_v.at[pl.ds(0, _RQ)],
                            acc.at[pl.ds(sid * _RPT + q * _RQ, _RQ)])
        plsc.subcore_barrier()

        base = sid * _EPT

        def chunk(i, _):
            e0 = base + i * _GC
            pltpu.sync_copy(dst_hbm.at[pl.ds(e0, _GC)], idx_v)
            pltpu.sync_copy(m_hbm.at[pl.ds(e0, _GC)], m_v)
            pltpu.sync_copy(m_v, acc.at[idx_v], add=True)
            return 0

        lax.fori_loop(0, _SFULL, chunk, 0)
        e0 = base + _SFULL * _GC
        pltpu.sync_copy(dst_hbm.at[pl.ds(e0, _STAIL)], idxt_v)
        pltpu.sync_copy(m_hbm.at[pl.ds(e0, _STAIL)], mt_v)
        pltpu.sync_copy(mt_v, acc.at[idxt_v], add=True)
        plsc.subcore_barrier()
        for q in range(_RPT // _RQ):
            r0 = sid * _RPT + q * _RQ
            pltpu.sync_copy(acc.at[pl.ds(r0, _RQ)], m_v.at[pl.ds(0, _RQ)])
            pltpu.sync_copy(m_v.at[pl.ds(0, _RQ)], y_hbm.at[pl.ds(r0, _RQ)])
        plsc.subcore_barrier()

    for ci in range(NC):
        @pl.when(cid == ci)
        def _():
            one_pass(m_refs[2 * ci], y_refs[2 * ci])
            one_pass(m_refs[2 * ci + 1], y_refs[2 * ci + 1])


def _f_body(x_ref, y0_ref, y1_ref, y2_ref, y3_ref,
            wz, bz, wd0, bd0, wd1,
            au0, au1, av0, av1, ao0, ao1, abo0,
            bu0, bu1, bv0, bv1, bo0w, bbo0,
            out_ref):
    x = x_ref[...]
    z = _dot(x, wz[...]) + bz[...]
    yy0 = y0_ref[...] + z
    yd0 = _dot(yy0, wd0[...]) + bd0[...]
    yd1 = _dot(y1_ref[...], wd1[...])
    yd2 = _dot(y2_ref[...], wd1[...])
    yd3 = _dot(y3_ref[...], wd1[...])

    u0 = _dot(yd0, au0[...])
    u1 = _dot(yd1, au1[...]); u2 = _dot(yd2, au1[...]); u3 = _dot(yd3, au1[...])
    v0 = _dot(yd0, av0[...])
    v1 = _dot(yd1, av1[...]); v2 = _dot(yd2, av1[...]); v3 = _dot(yd3, av1[...])
    t0 = u0 * v0 + u1 * v1 + u2 * v2 + u3 * v3
    o0 = _dot(t0, ao0[...]) + abo0[...]
    c1 = u2 * v3 - u3 * v2
    c2 = u3 * v1 - u1 * v3
    c3 = u1 * v2 - u2 * v1
    t1 = u0 * v1 + u1 * v0 + c1
    t2 = u0 * v2 + u2 * v0 + c2
    t3 = u0 * v3 + u3 * v0 + c3
    p0 = yd0 + o0
    p1 = yd1 + _dot(t1, ao1[...])
    p2 = yd2 + _dot(t2, ao1[...])
    p3 = yd3 + _dot(t3, ao1[...])

    q0 = _dot(p0, bu0[...])
    q1 = _dot(p1, bu1[...]); q2 = _dot(p2, bu1[...]); q3 = _dot(p3, bu1[...])
    r0 = _dot(p0, bv0[...])
    r1 = _dot(p1, bv1[...]); r2 = _dot(p2, bv1[...]); r3 = _dot(p3, bv1[...])
    s0 = q0 * r0 + q1 * r1 + q2 * r2 + q3 * r3
    oo0 = _dot(s0, bo0w[...]) + bbo0[...]
    out_ref[...] = p0 + oo0


def _final_dense(x, y0, y1, y2, y3, params):
    full = lambda: pl.BlockSpec((128, 128), lambda i: (0, 0))
    row = lambda: pl.BlockSpec((1, 128), lambda i: (0, 0))
    nb = pl.BlockSpec((_CN, 128), lambda i: (i, 0))
    a = params['itp0']; b = params['itp1']
    return pl.pallas_call(
        _f_body,
        grid=(_GN,),
        in_specs=[nb, nb, nb, nb, nb,
                  full(), row(), full(), row(), full(),
                  full(), full(), full(), full(), full(), full(), row(),
                  full(), full(), full(), full(), full(), row()],
        out_specs=nb,
        out_shape=jax.ShapeDtypeStruct((N, 128), jnp.float32),
    )(x, y0, y1, y2, y3,
      params['Wz'], params['bz'][None, :], params['Wd0'], params['bd0'][None, :],
      params['Wd1'],
      a['Wu0'], a['Wu1'], a['Wv0'], a['Wv1'], a['Wo0'], a['Wo1'], a['bo0'][None, :],
      b['Wu0'], b['Wu1'], b['Wv0'], b['Wv1'], b['Wo0'], b['bo0'][None, :])


def kernel(positions, atomic_numbers, dst_idx, src_idx, num_unpaired_electrons,
           total_charge, batch_segments, graph_mask, params):
    pos128 = jnp.pad(positions.astype(jnp.float32), ((0, 0), (0, 125)))
    an2 = atomic_numbers.astype(jnp.int32)[:, None]
    seg2 = batch_segments.astype(jnp.int32)[:, None]
    psiQ = total_charge.astype(jnp.float32)[:, None]
    psiS = num_unpaired_electrons.astype(jnp.float32)[:, None]
    maskf = graph_mask.astype(jnp.float32)[:, None]
    embed_p = jnp.pad(params['embed'], ((0, 128 - params['embed'].shape[0]), (0, 0)))
    dsti = dst_idx.astype(jnp.int32)
    srci = src_idx.astype(jnp.int32)

    wgtQ, wgtS, denQ, denS = _node_logits(an2, seg2, embed_p,
                                          params['Q'], params['S'], psiQ, psiS)
    x = _node_features(an2, seg2, embed_p, params['Q'], params['S'],
                       wgtQ, wgtS, denQ, denS, psiQ, psiS, maskf)
    disp_flat, xs = _sc_gather(pos128, x, dsti, srci)
    m0, m1, m2, m3 = _edge_messages(disp_flat.reshape(E, 16), xs, params['W_mp'])
    y0, y1, y2, y3 = _sc_scatter(m0, m1, m2, m3, dsti)
    out = _final_dense(x, y0, y1, y2, y3, params)
    return out[:, None, None, :]


# speedup vs baseline: 3.0735x; 1.1741x over previous
import functools
import math

import jax
import jax.numpy as jnp
import numpy as np
from jax import lax
from jax.experimental import pallas as pl
from jax.experimental.pallas import tpu as pltpu
from jax.experimental.pallas import tpu_sc as plsc

N = 10000
E = 160000
F = 128
K = 32
G = 64
CUTOFF = 5.0

NC = 2
NS = 16
NW = NC * NS

_HIGH = jax.lax.Precision.HIGHEST


def _dot(a, b):
    return jnp.dot(a, b, precision=_HIGH, preferred_element_type=jnp.float32)


_CN = 1000
_GN = N // _CN


def _b1_body(an_ref, seg_ref, embed_ref, wq_q, kp_q, kn_q, psi_q,
             wq_s, kp_s, kn_s, psi_s,
             wgtq_ref, wgts_ref, denq_ref, dens_ref):
    i = pl.program_id(0)
    an = an_ref[...]
    onez = (an == lax.broadcasted_iota(jnp.int32, (_CN, 128), 1)).astype(jnp.float32)
    e_z = _dot(onez, embed_ref[...])
    seg = seg_ref[...]
    segoh = (seg == lax.broadcasted_iota(jnp.int32, (_CN, G), 1)).astype(jnp.float32)

    @pl.when(i == 0)
    def _():
        denq_ref[...] = jnp.zeros_like(denq_ref)
        dens_ref[...] = jnp.zeros_like(dens_ref)

    def logits(wq, kp, kn, psi):
        q = _dot(e_z, wq[...])
        psi_at = _dot(segoh, psi[...])
        mpos = (psi_at >= 0.0).astype(jnp.float32)
        k_at = mpos * kp[...] + (1.0 - mpos) * kn[...]
        s = jnp.sum(q * k_at, axis=1, keepdims=True) * (1.0 / np.sqrt(F))
        return jax.nn.softplus(s)

    wgt_q = logits(wq_q, kp_q, kn_q, psi_q)
    wgt_s = logits(wq_s, kp_s, kn_s, psi_s)
    wgtq_ref[...] = wgt_q
    wgts_ref[...] = wgt_s
    denq_ref[...] += lax.dot_general(segoh, wgt_q, (((0,), (0,)), ((), ())),
                                     precision=_HIGH, preferred_element_type=jnp.float32)
    dens_ref[...] += lax.dot_general(segoh, wgt_s, (((0,), (0,)), ((), ())),
                                     precision=_HIGH, preferred_element_type=jnp.float32)


def _node_logits(an2, seg2, embed_p, pQ, pS, psiQ, psiS):
    full = lambda s: pl.BlockSpec(s, lambda i: (0, 0))
    return pl.pallas_call(
        _b1_body,
        grid=(_GN,),
        in_specs=[
            pl.BlockSpec((_CN, 1), lambda i: (i, 0)),
            pl.BlockSpec((_CN, 1), lambda i: (i, 0)),
            full((128, 128)),
            full((128, 128)), full((1, 128)), full((1, 128)), full((G, 1)),
            full((128, 128)), full((1, 128)), full((1, 128)), full((G, 1)),
        ],
        out_specs=[
            pl.BlockSpec((_CN, 1), lambda i: (i, 0)),
            pl.BlockSpec((_CN, 1), lambda i: (i, 0)),
            full((G, 1)), full((G, 1)),
        ],
        out_shape=[
            jax.ShapeDtypeStruct((N, 1), jnp.float32),
            jax.ShapeDtypeStruct((N, 1), jnp.float32),
            jax.ShapeDtypeStruct((G, 1), jnp.float32),
            jax.ShapeDtypeStruct((G, 1), jnp.float32),
        ],
    )(an2, seg2, embed_p, pQ['Wq'], pQ['k_pos'][None, :], pQ['k_neg'][None, :], psiQ,
      pS['Wq'], pS['k_pos'][None, :], pS['k_neg'][None, :], psiS)


def _b2_body(an_ref, seg_ref, embed_ref, wv_q, wv_s, wgtq_ref, wgts_ref,
             denq_ref, dens_ref, psiq_ref, psis_ref, maskf_ref, x_ref):
    an = an_ref[...]
    onez = (an == lax.broadcasted_iota(jnp.int32, (_CN, 128), 1)).astype(jnp.float32)
    e_z = _dot(onez, embed_ref[...])
    seg = seg_ref[...]
    segoh = (seg == lax.broadcasted_iota(jnp.int32, (_CN, G), 1)).astype(jnp.float32)
    mask_at = _dot(segoh, maskf_ref[...])

    def deloc(wv, wgt, den, psi):
        den_at = _dot(segoh, den[...])
        psi_at = _dot(segoh, psi[...])
        wn = wgt[...] / (den_at + 1e-8)
        coef = psi_at * wn * mask_at
        return coef * _dot(e_z, wv[...])

    x_ref[...] = (e_z + deloc(wv_q, wgtq_ref, denq_ref, psiq_ref)
                  + deloc(wv_s, wgts_ref, dens_ref, psis_ref))


def _node_features(an2, seg2, embed_p, pQ, pS, wgtQ, wgtS, denQ, denS,
                   psiQ, psiS, maskf):
    full = lambda s: pl.BlockSpec(s, lambda i: (0, 0))
    return pl.pallas_call(
        _b2_body,
        grid=(_GN,),
        in_specs=[
            pl.BlockSpec((_CN, 1), lambda i: (i, 0)),
            pl.BlockSpec((_CN, 1), lambda i: (i, 0)),
            full((128, 128)), full((128, 128)), full((128, 128)),
            pl.BlockSpec((_CN, 1), lambda i: (i, 0)),
            pl.BlockSpec((_CN, 1), lambda i: (i, 0)),
            full((G, 1)), full((G, 1)), full((G, 1)), full((G, 1)), full((G, 1)),
        ],
        out_specs=pl.BlockSpec((_CN, 128), lambda i: (i, 0)),
        out_shape=jax.ShapeDtypeStruct((N, 128), jnp.float32),
    )(an2, seg2, embed_p, pQ['Wv'], pS['Wv'], wgtQ, wgtS, denQ, denS,
      psiQ, psiS, maskf)


_EPW = E // NW
_GC = 128
_GFULL = _EPW // _GC
_GTAIL = _EPW - _GFULL * _GC

_sc_mesh = plsc.VectorSubcoreMesh(core_axis_name="c", subcore_axis_name="s")


@functools.partial(
    pl.kernel,
    out_type=(
        jax.ShapeDtypeStruct((E * 16,), jnp.float32),
        jax.ShapeDtypeStruct((E, 128), jnp.float32),
    ),
    mesh=_sc_mesh,
    scratch_types=[
        pltpu.VMEM((_GC,), jnp.int32),
        pltpu.VMEM((_GC,), jnp.int32),
        pltpu.VMEM((_GC, 128), jnp.float32),
        pltpu.VMEM((_GC, 128), jnp.float32),
        pltpu.VMEM((_GC, 128), jnp.float32),
        pltpu.VMEM((_GC * 16,), jnp.float32),
        pltpu.VMEM((_GTAIL,), jnp.int32),
        pltpu.VMEM((_GTAIL,), jnp.int32),
        pltpu.VMEM((_GTAIL, 128), jnp.float32),
        pltpu.VMEM((_GTAIL, 128), jnp.float32),
        pltpu.VMEM((_GTAIL, 128), jnp.float32),
        pltpu.VMEM((_GTAIL * 16,), jnp.float32),
        pltpu.SemaphoreType.DMA,
    ],
)
def _sc_gather(pos_hbm, x_hbm, dst_hbm, src_hbm,
               disp_out, xs_out,
               idxd_v, idxs_v, pd_v, ps_v, xs_v, dv_v,
               idxd_t, idxs_t, pd_t, ps_t, xs_t, dv_t, sem):
    wid = lax.axis_index("s") * NC + lax.axis_index("c")
    base = wid * _EPW

    def chunk(i, _):
        e0 = base + i * _GC
        pltpu.sync_copy(dst_hbm.at[pl.ds(e0, _GC)], idxd_v)
        pltpu.sync_copy(src_hbm.at[pl.ds(e0, _GC)], idxs_v)
        c1 = pltpu.async_copy(pos_hbm.at[idxd_v], pd_v, sem)
        c2 = pltpu.async_copy(pos_hbm.at[idxs_v], ps_v, sem)
        c3 = pltpu.async_copy(x_hbm.at[idxs_v], xs_v, sem)
        c1.wait(); c2.wait(); c3.wait()

        def drow(j, _):
            dv_v[pl.ds(j * 16, 16)] = ps_v[j, pl.ds(0, 16)] - pd_v[j, pl.ds(0, 16)]
            return 0

        lax.fori_loop(0, _GC, drow, 0)
        pltpu.sync_copy(dv_v, disp_out.at[pl.ds(e0 * 16, _GC * 16)])
        pltpu.sync_copy(xs_v, xs_out.at[pl.ds(e0, _GC)])
        return 0

    lax.fori_loop(0, _GFULL, chunk, 0)
    e0 = base + _GFULL * _GC
    pltpu.sync_copy(dst_hbm.at[pl.ds(e0, _GTAIL)], idxd_t)
    pltpu.sync_copy(src_hbm.at[pl.ds(e0, _GTAIL)], idxs_t)
    c1 = pltpu.async_copy(pos_hbm.at[idxd_t], pd_t, sem)
    c2 = pltpu.async_copy(pos_hbm.at[idxs_t], ps_t, sem)
    c3 = pltpu.async_copy(x_hbm.at[idxs_t], xs_t, sem)
    c1.wait(); c2.wait(); c3.wait()

    def drow_t(j, _):
        dv_t[pl.ds(j * 16, 16)] = ps_t[j, pl.ds(0, 16)] - pd_t[j, pl.ds(0, 16)]
        return 0

    lax.fori_loop(0, _GTAIL, drow_t, 0)
    pltpu.sync_copy(dv_t, disp_out.at[pl.ds(e0 * 16, _GTAIL * 16)])
    pltpu.sync_copy(xs_t, xs_out.at[pl.ds(e0, _GTAIL)])


_CE = 640
_GE = E // _CE

_LOG_BINOM = np.array(
    [math.lgamma(K) - math.lgamma(k + 1.0) - math.lgamma(K - 1.0 - k + 1.0)
     for k in range(K)], dtype=np.float32)[None, :]
_KARR = np.arange(K, dtype=np.float32)[None, :]


def _d_body(dsp_ref, xs_ref, wmp_ref, lb_ref, ka_ref,
            m0_ref, m1_ref, m2_ref, m3_ref):
    disp = dsp_ref[...]
    r2 = jnp.sum(disp * disp, axis=1, keepdims=True) + 1e-12
    r = jnp.sqrt(r2)
    u = 1.0 / (1.0 + r)
    log_u = jnp.log(jnp.clip(u, 1e-10, 1.0))
    log_1mu = jnp.log(jnp.clip(1.0 - u, 1e-10, 1.0))
    kb = ka_ref[...]
    radial = jnp.exp(lb_ref[...] + kb * log_u + (K - 1.0 - kb) * log_1mu)
    xx = r * (1.0 / CUTOFF)
    x2 = jnp.clip(xx, 0.0, 1.0 - 1e-6) ** 2
    cut = jnp.where(xx < 1.0, jnp.exp(1.0 - 1.0 / (1.0 - x2)), 0.0)
    radial = radial * cut
    rp = _dot(radial, wmp_ref[...])
    g = rp * xs_ref[...]
    inv_r = 1.0 / r
    m0_ref[...] = g
    m1_ref[...] = (disp[:, 0:1] * inv_r) * g
    m2_ref[...] = (disp[:, 1:2] * inv_r) * g
    m3_ref[...] = (disp[:, 2:3] * inv_r) * g


def _edge_messages(dsp, xs, wmp):
    eb = lambda w: pl.BlockSpec((_CE, w), lambda i: (i, 0))
    return pl.pallas_call(
        _d_body,
        grid=(_GE,),
        in_specs=[eb(16), eb(128), pl.BlockSpec((32, 128), lambda i: (0, 0)),
                  pl.BlockSpec((1, 32), lambda i: (0, 0)),
                  pl.BlockSpec((1, 32), lambda i: (0, 0))],
        out_specs=[eb(128)] * 4,
        out_shape=[jax.ShapeDtypeStruct((E, 128), jnp.float32)] * 4,
    )(dsp, xs, wmp, jnp.asarray(_LOG_BINOM), jnp.asarray(_KARR))


_EPT = E // NS
_SFULL = _EPT // _GC
_STAIL = _EPT - _SFULL * _GC
NPAD = 10240
_RPT = NPAD // NS
_RQ = 128


@functools.partial(
    pl.kernel,
    out_type=tuple(jax.ShapeDtypeStruct((NPAD, 128), jnp.float32) for _ in range(4)),
    mesh=_sc_mesh,
    scratch_types=[
        pltpu.VMEM((_GC, 128), jnp.float32),
        pltpu.VMEM((_GC, 128), jnp.float32),
        pltpu.VMEM((_GC,), jnp.int32),
        pltpu.VMEM((_GC,), jnp.int32),
        pltpu.VMEM((_STAIL, 128), jnp.float32),
        pltpu.VMEM((_STAIL,), jnp.int32),
        pltpu.VMEM_SHARED((NPAD, 128), jnp.float32),
        pltpu.SemaphoreType.DMA,
        pltpu.SemaphoreType.DMA,
        pltpu.SemaphoreType.DMA,
        pltpu.SemaphoreType.DMA,
    ],
)
def _sc_scatter(m0_hbm, m1_hbm, m2_hbm, m3_hbm, dst_hbm,
                y0_hbm, y1_hbm, y2_hbm, y3_hbm,
                m_v0, m_v1, idx_v0, idx_v1,
                mt_v, idxt_v, acc,
                sm0, sm1, si0, si1):
    cid = lax.axis_index("c")
    sid = lax.axis_index("s")
    m_refs = (m0_hbm, m1_hbm, m2_hbm, m3_hbm)
    y_refs = (y0_hbm, y1_hbm, y2_hbm, y3_hbm)
    bufs = ((m_v0, idx_v0, sm0, si0), (m_v1, idx_v1, sm1, si1))

    def one_pass(m_hbm, y_hbm):
        zv = jnp.zeros((16,), jnp.float32)

        def zrow(i, _):
            for j in range(8):
                m_v0[i, pl.ds(j * 16, 16)] = zv
            return 0

        lax.fori_loop(0, _GC, zrow, 0)
        for q in range(_RPT // _RQ):
            pltpu.sync_copy(m_v0.at[pl.ds(0, _RQ)],
                            acc.at[pl.ds(sid * _RPT + q * _RQ, _RQ)])
        plsc.subcore_barrier()

        base = sid * _EPT

        def start(b, ci):
            mv, iv, sm, si = bufs[b]
            e0 = base + ci * _GC
            pltpu.async_copy(dst_hbm.at[pl.ds(e0, _GC)], iv, si)
            pltpu.async_copy(m_hbm.at[pl.ds(e0, _GC)], mv, sm)

        def finish(b):
            mv, iv, sm, si = bufs[b]
            pltpu.make_async_copy(dst_hbm.at[pl.ds(0, _GC)], iv, si).wait()
            pltpu.make_async_copy(m_hbm.at[pl.ds(0, _GC)], mv, sm).wait()
            pltpu.sync_copy(mv, acc.at[iv], add=True)

        for b in range(2):
            start(b, b)

        def ring(j, _):
            for b in range(2):
                ci = 2 * j + b
                finish(b)
                nxt = ci + 2

                @pl.when(nxt < _SFULL)
                def _():
                    start(b, nxt)
            return 0

        lax.fori_loop(0, _SFULL // 2, ring, 0)
        e0 = base + _SFULL * _GC
        pltpu.sync_copy(dst_hbm.at[pl.ds(e0, _STAIL)], idxt_v)
        pltpu.sync_copy(m_hbm.at[pl.ds(e0, _STAIL)], mt_v)
        pltpu.sync_copy(mt_v, acc.at[idxt_v], add=True)
        plsc.subcore_barrier()
        for q in range(_RPT // _RQ):
            r0 = sid * _RPT + q * _RQ
            pltpu.sync_copy(acc.at[pl.ds(r0, _RQ)], m_v0.at[pl.ds(0, _RQ)])
            pltpu.sync_copy(m_v0.at[pl.ds(0, _RQ)], y_hbm.at[pl.ds(r0, _RQ)])
        plsc.subcore_barrier()

    for ci in range(NC):
        @pl.when(cid == ci)
        def _():
            one_pass(m_refs[2 * ci], y_refs[2 * ci])
            one_pass(m_refs[2 * ci + 1], y_refs[2 * ci + 1])


def _f_body(x_ref, y0_ref, y1_ref, y2_ref, y3_ref,
            wz, bz, wd0, bd0, wd1,
            au0, au1, av0, av1, ao0, ao1, abo0,
            bu0, bu1, bv0, bv1, bo0w, bbo0,
            out_ref):
    x = x_ref[...]
    z = _dot(x, wz[...]) + bz[...]
    yy0 = y0_ref[...] + z
    yd0 = _dot(yy0, wd0[...]) + bd0[...]
    yd1 = _dot(y1_ref[...], wd1[...])
    yd2 = _dot(y2_ref[...], wd1[...])
    yd3 = _dot(y3_ref[...], wd1[...])

    u0 = _dot(yd0, au0[...])
    u1 = _dot(yd1, au1[...]); u2 = _dot(yd2, au1[...]); u3 = _dot(yd3, au1[...])
    v0 = _dot(yd0, av0[...])
    v1 = _dot(yd1, av1[...]); v2 = _dot(yd2, av1[...]); v3 = _dot(yd3, av1[...])
    t0 = u0 * v0 + u1 * v1 + u2 * v2 + u3 * v3
    o0 = _dot(t0, ao0[...]) + abo0[...]
    c1 = u2 * v3 - u3 * v2
    c2 = u3 * v1 - u1 * v3
    c3 = u1 * v2 - u2 * v1
    t1 = u0 * v1 + u1 * v0 + c1
    t2 = u0 * v2 + u2 * v0 + c2
    t3 = u0 * v3 + u3 * v0 + c3
    p0 = yd0 + o0
    p1 = yd1 + _dot(t1, ao1[...])
    p2 = yd2 + _dot(t2, ao1[...])
    p3 = yd3 + _dot(t3, ao1[...])

    q0 = _dot(p0, bu0[...])
    q1 = _dot(p1, bu1[...]); q2 = _dot(p2, bu1[...]); q3 = _dot(p3, bu1[...])
    r0 = _dot(p0, bv0[...])
    r1 = _dot(p1, bv1[...]); r2 = _dot(p2, bv1[...]); r3 = _dot(p3, bv1[...])
    s0 = q0 * r0 + q1 * r1 + q2 * r2 + q3 * r3
    oo0 = _dot(s0, bo0w[...]) + bbo0[...]
    out_ref[...] = p0 + oo0


def _final_dense(x, y0, y1, y2, y3, params):
    full = lambda: pl.BlockSpec((128, 128), lambda i: (0, 0))
    row = lambda: pl.BlockSpec((1, 128), lambda i: (0, 0))
    nb = pl.BlockSpec((_CN, 128), lambda i: (i, 0))
    a = params['itp0']; b = params['itp1']
    return pl.pallas_call(
        _f_body,
        grid=(_GN,),
        in_specs=[nb, nb, nb, nb, nb,
                  full(), row(), full(), row(), full(),
                  full(), full(), full(), full(), full(), full(), row(),
                  full(), full(), full(), full(), full(), row()],
        out_specs=nb,
        out_shape=jax.ShapeDtypeStruct((N, 128), jnp.float32),
    )(x, y0, y1, y2, y3,
      params['Wz'], params['bz'][None, :], params['Wd0'], params['bd0'][None, :],
      params['Wd1'],
      a['Wu0'], a['Wu1'], a['Wv0'], a['Wv1'], a['Wo0'], a['Wo1'], a['bo0'][None, :],
      b['Wu0'], b['Wu1'], b['Wv0'], b['Wv1'], b['Wo0'], b['bo0'][None, :])


def kernel(positions, atomic_numbers, dst_idx, src_idx, num_unpaired_electrons,
           total_charge, batch_segments, graph_mask, params):
    pos128 = jnp.pad(positions.astype(jnp.float32), ((0, 0), (0, 125)))
    an2 = atomic_numbers.astype(jnp.int32)[:, None]
    seg2 = batch_segments.astype(jnp.int32)[:, None]
    psiQ = total_charge.astype(jnp.float32)[:, None]
    psiS = num_unpaired_electrons.astype(jnp.float32)[:, None]
    maskf = graph_mask.astype(jnp.float32)[:, None]
    embed_p = jnp.pad(params['embed'], ((0, 128 - params['embed'].shape[0]), (0, 0)))
    dsti = dst_idx.astype(jnp.int32)
    srci = src_idx.astype(jnp.int32)

    wgtQ, wgtS, denQ, denS = _node_logits(an2, seg2, embed_p,
                                          params['Q'], params['S'], psiQ, psiS)
    x = _node_features(an2, seg2, embed_p, params['Q'], params['S'],
                       wgtQ, wgtS, denQ, denS, psiQ, psiS, maskf)
    disp_flat, xs = _sc_gather(pos128, x, dsti, srci)
    m0, m1, m2, m3 = _edge_messages(disp_flat.reshape(E, 16), xs, params['W_mp'])
    y0, y1, y2, y3 = _sc_scatter(m0, m1, m2, m3, dsti)
    out = _final_dense(x, y0, y1, y2, y3, params)
    return out[:, None, None, :]


# speedup vs baseline: 3.3252x; 1.0819x over previous
import functools
import math

import jax
import jax.numpy as jnp
import numpy as np
from jax import lax
from jax.experimental import pallas as pl
from jax.experimental.pallas import tpu as pltpu
from jax.experimental.pallas import tpu_sc as plsc

N = 10000
E = 160000
F = 128
K = 32
G = 64
CUTOFF = 5.0

NC = 2
NS = 16
NW = NC * NS

_HIGH = jax.lax.Precision.HIGHEST


def _dot(a, b):
    return jnp.dot(a, b, precision=_HIGH, preferred_element_type=jnp.float32)


_CN = 1000
_GN = N // _CN


def _b1_body(an_ref, seg_ref, embed_ref, wq_q, kp_q, kn_q, psi_q,
             wq_s, kp_s, kn_s, psi_s,
             wgtq_ref, wgts_ref, denq_ref, dens_ref):
    i = pl.program_id(0)
    an = an_ref[...]
    onez = (an == lax.broadcasted_iota(jnp.int32, (_CN, 128), 1)).astype(jnp.float32)
    e_z = _dot(onez, embed_ref[...])
    seg = seg_ref[...]
    segoh = (seg == lax.broadcasted_iota(jnp.int32, (_CN, G), 1)).astype(jnp.float32)

    @pl.when(i == 0)
    def _():
        denq_ref[...] = jnp.zeros_like(denq_ref)
        dens_ref[...] = jnp.zeros_like(dens_ref)

    qq = _dot(e_z, jnp.concatenate([wq_q[...], wq_s[...]], axis=1))

    def logits(q, kp, kn, psi):
        psi_at = _dot(segoh, psi[...])
        mpos = (psi_at >= 0.0).astype(jnp.float32)
        k_at = mpos * kp[...] + (1.0 - mpos) * kn[...]
        s = jnp.sum(q * k_at, axis=1, keepdims=True) * (1.0 / np.sqrt(F))
        return jax.nn.softplus(s)

    wgt_q = logits(qq[:, :128], kp_q, kn_q, psi_q)
    wgt_s = logits(qq[:, 128:], kp_s, kn_s, psi_s)
    wgtq_ref[...] = wgt_q
    wgts_ref[...] = wgt_s
    denq_ref[...] += lax.dot_general(segoh, wgt_q, (((0,), (0,)), ((), ())),
                                     precision=_HIGH, preferred_element_type=jnp.float32)
    dens_ref[...] += lax.dot_general(segoh, wgt_s, (((0,), (0,)), ((), ())),
                                     precision=_HIGH, preferred_element_type=jnp.float32)


def _node_logits(an2, seg2, embed_p, pQ, pS, psiQ, psiS):
    full = lambda s: pl.BlockSpec(s, lambda i: (0, 0))
    return pl.pallas_call(
        _b1_body,
        grid=(_GN,),
        in_specs=[
            pl.BlockSpec((_CN, 1), lambda i: (i, 0)),
            pl.BlockSpec((_CN, 1), lambda i: (i, 0)),
            full((128, 128)),
            full((128, 128)), full((1, 128)), full((1, 128)), full((G, 1)),
            full((128, 128)), full((1, 128)), full((1, 128)), full((G, 1)),
        ],
        out_specs=[
            pl.BlockSpec((_CN, 1), lambda i: (i, 0)),
            pl.BlockSpec((_CN, 1), lambda i: (i, 0)),
            full((G, 1)), full((G, 1)),
        ],
        out_shape=[
            jax.ShapeDtypeStruct((N, 1), jnp.float32),
            jax.ShapeDtypeStruct((N, 1), jnp.float32),
            jax.ShapeDtypeStruct((G, 1), jnp.float32),
            jax.ShapeDtypeStruct((G, 1), jnp.float32),
        ],
    )(an2, seg2, embed_p, pQ['Wq'], pQ['k_pos'][None, :], pQ['k_neg'][None, :], psiQ,
      pS['Wq'], pS['k_pos'][None, :], pS['k_neg'][None, :], psiS)


def _b2_body(an_ref, seg_ref, embed_ref, wv_q, wv_s, wgtq_ref, wgts_ref,
             denq_ref, dens_ref, psiq_ref, psis_ref, maskf_ref, x_ref):
    an = an_ref[...]
    onez = (an == lax.broadcasted_iota(jnp.int32, (_CN, 128), 1)).astype(jnp.float32)
    e_z = _dot(onez, embed_ref[...])
    seg = seg_ref[...]
    segoh = (seg == lax.broadcasted_iota(jnp.int32, (_CN, G), 1)).astype(jnp.float32)
    mask_at = _dot(segoh, maskf_ref[...])
    ev = _dot(e_z, jnp.concatenate([wv_q[...], wv_s[...]], axis=1))

    def deloc(evp, wgt, den, psi):
        den_at = _dot(segoh, den[...])
        psi_at = _dot(segoh, psi[...])
        wn = wgt[...] / (den_at + 1e-8)
        coef = psi_at * wn * mask_at
        return coef * evp

    x_ref[...] = (e_z + deloc(ev[:, :128], wgtq_ref, denq_ref, psiq_ref)
                  + deloc(ev[:, 128:], wgts_ref, dens_ref, psis_ref))


def _node_features(an2, seg2, embed_p, pQ, pS, wgtQ, wgtS, denQ, denS,
                   psiQ, psiS, maskf):
    full = lambda s: pl.BlockSpec(s, lambda i: (0, 0))
    return pl.pallas_call(
        _b2_body,
        grid=(_GN,),
        in_specs=[
            pl.BlockSpec((_CN, 1), lambda i: (i, 0)),
            pl.BlockSpec((_CN, 1), lambda i: (i, 0)),
            full((128, 128)), full((128, 128)), full((128, 128)),
            pl.BlockSpec((_CN, 1), lambda i: (i, 0)),
            pl.BlockSpec((_CN, 1), lambda i: (i, 0)),
            full((G, 1)), full((G, 1)), full((G, 1)), full((G, 1)), full((G, 1)),
        ],
        out_specs=pl.BlockSpec((_CN, 128), lambda i: (i, 0)),
        out_shape=jax.ShapeDtypeStruct((N, 128), jnp.float32),
    )(an2, seg2, embed_p, pQ['Wv'], pS['Wv'], wgtQ, wgtS, denQ, denS,
      psiQ, psiS, maskf)


_EPW = E // NW
_GC = 128
_GFULL = _EPW // _GC
_GTAIL = _EPW - _GFULL * _GC

_sc_mesh = plsc.VectorSubcoreMesh(core_axis_name="c", subcore_axis_name="s")


@functools.partial(
    pl.kernel,
    out_type=(
        jax.ShapeDtypeStruct((E * 16,), jnp.float32),
        jax.ShapeDtypeStruct((E, 128), jnp.float32),
    ),
    mesh=_sc_mesh,
    scratch_types=[
        pltpu.VMEM((_GC,), jnp.int32),
        pltpu.VMEM((_GC,), jnp.int32),
        pltpu.VMEM((_GC, 128), jnp.float32),
        pltpu.VMEM((_GC, 128), jnp.float32),
        pltpu.VMEM((_GC, 128), jnp.float32),
        pltpu.VMEM((_GC * 16,), jnp.float32),
        pltpu.VMEM((_GTAIL,), jnp.int32),
        pltpu.VMEM((_GTAIL,), jnp.int32),
        pltpu.VMEM((_GTAIL, 128), jnp.float32),
        pltpu.VMEM((_GTAIL, 128), jnp.float32),
        pltpu.VMEM((_GTAIL, 128), jnp.float32),
        pltpu.VMEM((_GTAIL * 16,), jnp.float32),
        pltpu.SemaphoreType.DMA,
    ],
)
def _sc_gather(pos_hbm, x_hbm, dst_hbm, src_hbm,
               disp_out, xs_out,
               idxd_v, idxs_v, pd_v, ps_v, xs_v, dv_v,
               idxd_t, idxs_t, pd_t, ps_t, xs_t, dv_t, sem):
    wid = lax.axis_index("s") * NC + lax.axis_index("c")
    base = wid * _EPW

    def chunk(i, _):
        e0 = base + i * _GC
        pltpu.sync_copy(dst_hbm.at[pl.ds(e0, _GC)], idxd_v)
        pltpu.sync_copy(src_hbm.at[pl.ds(e0, _GC)], idxs_v)
        c1 = pltpu.async_copy(pos_hbm.at[idxd_v], pd_v, sem)
        c2 = pltpu.async_copy(pos_hbm.at[idxs_v], ps_v, sem)
        c3 = pltpu.async_copy(x_hbm.at[idxs_v], xs_v, sem)
        c1.wait(); c2.wait(); c3.wait()

        def drow(j, _):
            dv_v[pl.ds(j * 16, 16)] = ps_v[j, pl.ds(0, 16)] - pd_v[j, pl.ds(0, 16)]
            return 0

        lax.fori_loop(0, _GC, drow, 0)
        pltpu.sync_copy(dv_v, disp_out.at[pl.ds(e0 * 16, _GC * 16)])
        pltpu.sync_copy(xs_v, xs_out.at[pl.ds(e0, _GC)])
        return 0

    lax.fori_loop(0, _GFULL, chunk, 0)
    e0 = base + _GFULL * _GC
    pltpu.sync_copy(dst_hbm.at[pl.ds(e0, _GTAIL)], idxd_t)
    pltpu.sync_copy(src_hbm.at[pl.ds(e0, _GTAIL)], idxs_t)
    c1 = pltpu.async_copy(pos_hbm.at[idxd_t], pd_t, sem)
    c2 = pltpu.async_copy(pos_hbm.at[idxs_t], ps_t, sem)
    c3 = pltpu.async_copy(x_hbm.at[idxs_t], xs_t, sem)
    c1.wait(); c2.wait(); c3.wait()

    def drow_t(j, _):
        dv_t[pl.ds(j * 16, 16)] = ps_t[j, pl.ds(0, 16)] - pd_t[j, pl.ds(0, 16)]
        return 0

    lax.fori_loop(0, _GTAIL, drow_t, 0)
    pltpu.sync_copy(dv_t, disp_out.at[pl.ds(e0 * 16, _GTAIL * 16)])
    pltpu.sync_copy(xs_t, xs_out.at[pl.ds(e0, _GTAIL)])


_CE = 1600
_GE = E // _CE

_LOG_BINOM = np.array(
    [math.lgamma(K) - math.lgamma(k + 1.0) - math.lgamma(K - 1.0 - k + 1.0)
     for k in range(K)], dtype=np.float32)[None, :]
_KARR = np.arange(K, dtype=np.float32)[None, :]


def _d_body(dsp_ref, xs_ref, wmp_ref, lb_ref, ka_ref,
            m0_ref, m1_ref, m2_ref, m3_ref):
    disp = dsp_ref[...]
    r2 = jnp.sum(disp * disp, axis=1, keepdims=True) + 1e-12
    r = jnp.sqrt(r2)
    u = 1.0 / (1.0 + r)
    log_u = jnp.log(jnp.clip(u, 1e-10, 1.0))
    log_1mu = jnp.log(jnp.clip(1.0 - u, 1e-10, 1.0))
    kb = ka_ref[...]
    radial = jnp.exp(lb_ref[...] + kb * log_u + (K - 1.0 - kb) * log_1mu)
    xx = r * (1.0 / CUTOFF)
    x2 = jnp.clip(xx, 0.0, 1.0 - 1e-6) ** 2
    cut = jnp.where(xx < 1.0, jnp.exp(1.0 - 1.0 / (1.0 - x2)), 0.0)
    radial = radial * cut
    rp = _dot(radial, wmp_ref[...])
    g = rp * xs_ref[...]
    inv_r = 1.0 / r
    m0_ref[...] = g
    m1_ref[...] = (disp[:, 0:1] * inv_r) * g
    m2_ref[...] = (disp[:, 1:2] * inv_r) * g
    m3_ref[...] = (disp[:, 2:3] * inv_r) * g


def _edge_messages(dsp, xs, wmp):
    eb = lambda w: pl.BlockSpec((_CE, w), lambda i: (i, 0))
    return pl.pallas_call(
        _d_body,
        grid=(_GE,),
        in_specs=[eb(16),
                  eb(128), pl.BlockSpec((32, 128), lambda i: (0, 0)),
                  pl.BlockSpec((1, 32), lambda i: (0, 0)),
                  pl.BlockSpec((1, 32), lambda i: (0, 0))],
        out_specs=[eb(128)] * 4,
        out_shape=[jax.ShapeDtypeStruct((E, 128), jnp.float32)] * 4,
    )(dsp.reshape(E, 16), xs, wmp,
      jnp.asarray(_LOG_BINOM), jnp.asarray(_KARR))


_EPT = E // NS
_SFULL = _EPT // _GC
_STAIL = _EPT - _SFULL * _GC
NPAD = 10240
_RPT = NPAD // NS
_RQ = 128


@functools.partial(
    pl.kernel,
    out_type=tuple(jax.ShapeDtypeStruct((NPAD, 128), jnp.float32) for _ in range(4)),
    mesh=_sc_mesh,
    scratch_types=[
        pltpu.VMEM((_GC, 128), jnp.float32),
        pltpu.VMEM((_GC, 128), jnp.float32),
        pltpu.VMEM((_GC,), jnp.int32),
        pltpu.VMEM((_GC,), jnp.int32),
        pltpu.VMEM((_STAIL, 128), jnp.float32),
        pltpu.VMEM((_STAIL,), jnp.int32),
        pltpu.VMEM_SHARED((NPAD, 128), jnp.float32),
        pltpu.SemaphoreType.DMA,
        pltpu.SemaphoreType.DMA,
        pltpu.SemaphoreType.DMA,
        pltpu.SemaphoreType.DMA,
    ],
)
def _sc_scatter(m0_hbm, m1_hbm, m2_hbm, m3_hbm, dst_hbm,
                y0_hbm, y1_hbm, y2_hbm, y3_hbm,
                m_v0, m_v1, idx_v0, idx_v1,
                mt_v, idxt_v, acc,
                sm0, sm1, si0, si1):
    cid = lax.axis_index("c")
    sid = lax.axis_index("s")
    m_refs = (m0_hbm, m1_hbm, m2_hbm, m3_hbm)
    y_refs = (y0_hbm, y1_hbm, y2_hbm, y3_hbm)
    bufs = ((m_v0, idx_v0, sm0, si0), (m_v1, idx_v1, sm1, si1))

    def one_pass(m_hbm, y_hbm):
        zv = jnp.zeros((16,), jnp.float32)

        def zrow(i, _):
            for j in range(8):
                m_v0[i, pl.ds(j * 16, 16)] = zv
            return 0

        lax.fori_loop(0, _GC, zrow, 0)
        for q in range(_RPT // _RQ):
            pltpu.sync_copy(m_v0.at[pl.ds(0, _RQ)],
                            acc.at[pl.ds(sid * _RPT + q * _RQ, _RQ)])
        plsc.subcore_barrier()

        base = sid * _EPT

        def start(b, ci):
            mv, iv, sm, si = bufs[b]
            e0 = base + ci * _GC
            pltpu.async_copy(dst_hbm.at[pl.ds(e0, _GC)], iv, si)
            pltpu.async_copy(m_hbm.at[pl.ds(e0, _GC)], mv, sm)

        def finish(b):
            mv, iv, sm, si = bufs[b]
            pltpu.make_async_copy(dst_hbm.at[pl.ds(0, _GC)], iv, si).wait()
            pltpu.make_async_copy(m_hbm.at[pl.ds(0, _GC)], mv, sm).wait()
            pltpu.sync_copy(mv, acc.at[iv], add=True)

        for b in range(2):
            start(b, b)

        def ring(j, _):
            for b in range(2):
                ci = 2 * j + b
                finish(b)
                nxt = ci + 2

                @pl.when(nxt < _SFULL)
                def _():
                    start(b, nxt)
            return 0

        lax.fori_loop(0, _SFULL // 2, ring, 0)
        e0 = base + _SFULL * _GC
        pltpu.sync_copy(dst_hbm.at[pl.ds(e0, _STAIL)], idxt_v)
        pltpu.sync_copy(m_hbm.at[pl.ds(e0, _STAIL)], mt_v)
        pltpu.sync_copy(mt_v, acc.at[idxt_v], add=True)
        plsc.subcore_barrier()
        for q in range(_RPT // _RQ):
            r0 = sid * _RPT + q * _RQ
            pltpu.sync_copy(acc.at[pl.ds(r0, _RQ)], m_v0.at[pl.ds(0, _RQ)])
            pltpu.sync_copy(m_v0.at[pl.ds(0, _RQ)], y_hbm.at[pl.ds(r0, _RQ)])
        plsc.subcore_barrier()

    for ci in range(NC):
        @pl.when(cid == ci)
        def _():
            one_pass(m_refs[2 * ci], y_refs[2 * ci])
            one_pass(m_refs[2 * ci + 1], y_refs[2 * ci + 1])


def _f_body(x_ref, y0_ref, y1_ref, y2_ref, y3_ref,
            wz, bz, wd0, bd0, wd1,
            auv0, auv1, ao0, ao1, abo0,
            buv0, buv1, bo0w, bbo0,
            out_ref):
    x = x_ref[...]
    z = _dot(x, wz[...]) + bz[...]
    yy0 = y0_ref[...] + z
    yd0 = _dot(yy0, wd0[...]) + bd0[...]
    y123 = jnp.concatenate([y1_ref[...], y2_ref[...], y3_ref[...]], axis=0)
    yd123 = _dot(y123, wd1[...])

    uv0 = _dot(yd0, auv0[...])
    uv123 = _dot(yd123, auv1[...])
    u0 = uv0[:, :128]; v0 = uv0[:, 128:]
    u1 = uv123[:_CN, :128]; v1 = uv123[:_CN, 128:]
    u2 = uv123[_CN:2 * _CN, :128]; v2 = uv123[_CN:2 * _CN, 128:]
    u3 = uv123[2 * _CN:, :128]; v3 = uv123[2 * _CN:, 128:]
    t0 = u0 * v0 + u1 * v1 + u2 * v2 + u3 * v3
    o0 = _dot(t0, ao0[...]) + abo0[...]
    t1 = u0 * v1 + u1 * v0 + (u2 * v3 - u3 * v2)
    t2 = u0 * v2 + u2 * v0 + (u3 * v1 - u1 * v3)
    t3 = u0 * v3 + u3 * v0 + (u1 * v2 - u2 * v1)
    t123 = jnp.concatenate([t1, t2, t3], axis=0)
    o123 = _dot(t123, ao1[...])
    p0 = yd0 + o0
    p123 = yd123 + o123

    qr0 = _dot(p0, buv0[...])
    qr123 = _dot(p123, buv1[...])
    s0 = (qr0[:, :128] * qr0[:, 128:]
          + qr123[:_CN, :128] * qr123[:_CN, 128:]
          + qr123[_CN:2 * _CN, :128] * qr123[_CN:2 * _CN, 128:]
          + qr123[2 * _CN:, :128] * qr123[2 * _CN:, 128:])
    oo0 = _dot(s0, bo0w[...]) + bbo0[...]
    out_ref[...] = p0 + oo0


def _final_dense(x, y0, y1, y2, y3, params):
    full = lambda: pl.BlockSpec((128, 128), lambda i: (0, 0))
    wide = lambda: pl.BlockSpec((128, 256), lambda i: (0, 0))
    row = lambda: pl.BlockSpec((1, 128), lambda i: (0, 0))
    nb = pl.BlockSpec((_CN, 128), lambda i: (i, 0))
    a = params['itp0']; b = params['itp1']
    auv0 = jnp.concatenate([a['Wu0'], a['Wv0']], axis=1)
    auv1 = jnp.concatenate([a['Wu1'], a['Wv1']], axis=1)
    buv0 = jnp.concatenate([b['Wu0'], b['Wv0']], axis=1)
    buv1 = jnp.concatenate([b['Wu1'], b['Wv1']], axis=1)
    return pl.pallas_call(
        _f_body,
        grid=(_GN,),
        in_specs=[nb, nb, nb, nb, nb,
                  full(), row(), full(), row(), full(),
                  wide(), wide(), full(), full(), row(),
                  wide(), wide(), full(), row()],
        out_specs=nb,
        out_shape=jax.ShapeDtypeStruct((N, 128), jnp.float32),
    )(x, y0, y1, y2, y3,
      params['Wz'], params['bz'][None, :], params['Wd0'], params['bd0'][None, :],
      params['Wd1'],
      auv0, auv1, a['Wo0'], a['Wo1'], a['bo0'][None, :],
      buv0, buv1, b['Wo0'], b['bo0'][None, :])


def kernel(positions, atomic_numbers, dst_idx, src_idx, num_unpaired_electrons,
           total_charge, batch_segments, graph_mask, params):
    pos128 = jnp.pad(positions.astype(jnp.float32), ((0, 0), (0, 125)))
    an2 = atomic_numbers.astype(jnp.int32)[:, None]
    seg2 = batch_segments.astype(jnp.int32)[:, None]
    psiQ = total_charge.astype(jnp.float32)[:, None]
    psiS = num_unpaired_electrons.astype(jnp.float32)[:, None]
    maskf = graph_mask.astype(jnp.float32)[:, None]
    embed_p = jnp.pad(params['embed'], ((0, 128 - params['embed'].shape[0]), (0, 0)))
    dsti = dst_idx.astype(jnp.int32)
    srci = src_idx.astype(jnp.int32)

    wgtQ, wgtS, denQ, denS = _node_logits(an2, seg2, embed_p,
                                          params['Q'], params['S'], psiQ, psiS)
    x = _node_features(an2, seg2, embed_p, params['Q'], params['S'],
                       wgtQ, wgtS, denQ, denS, psiQ, psiS, maskf)
    disp_flat, xs = _sc_gather(pos128, x, dsti, srci)
    m0, m1, m2, m3 = _edge_messages(disp_flat, xs, params['W_mp'])
    y0, y1, y2, y3 = _sc_scatter(m0, m1, m2, m3, dsti)
    out = _final_dense(x, y0, y1, y2, y3, params)
    return out[:, None, None, :]
